# Initial kernel scaffold; baseline (speedup 1.0000x reference)
import jax
import jax.numpy as jnp
from jax.experimental import pallas as pl


def kernel(x, edge_index, Wl0, bl0, Wr0, g0, b0, Wl1, bl1, Wr1, g1, b1, Wl2, bl2, Wr2, g2, b2, Wm1, bm1, Wm2, bm2, Wm3, bm3):
    raise NotImplementedError("write your pallas kernel here")


# speedup vs baseline: 3.2409x; 3.2409x over previous
import functools

import jax
import jax.numpy as jnp
from jax import lax
from jax.experimental import pallas as pl
from jax.experimental.pallas import tpu as pltpu
from jax.experimental.pallas import tpu_sc as plsc

N = 10000
E = 320000
D = 128
H = 256

NC = 2
NS = 16
NW = NC * NS
EPW = E // NW
CH = 80
NCHUNK = EPW // CH
RPT = 624
REM = N - NS * RPT


@functools.lru_cache(maxsize=None)
def _get_mesh():
    return plsc.VectorSubcoreMesh(core_axis_name="c", subcore_axis_name="s",
                                  num_cores=NC, num_subcores=NS)


def _zero_shared(sid, zeros_hbm, sh):
    pltpu.sync_copy(zeros_hbm, sh.at[pl.ds(sid * RPT, RPT)])

    @pl.when(sid == NS - 1)
    def _():
        pltpu.sync_copy(zeros_hbm.at[pl.ds(0, REM)], sh.at[pl.ds(NS * RPT, REM)])


def _drain_shared(cid, sid, sh, out):
    pltpu.sync_copy(sh.at[pl.ds(sid * RPT, RPT)],
                    out.at[cid, pl.ds(sid * RPT, RPT)])

    @pl.when(sid == NS - 1)
    def _():
        pltpu.sync_copy(sh.at[pl.ds(NS * RPT, REM)],
                        out.at[cid, pl.ds(NS * RPT, REM)])


def _seg_kernel(x_hbm, src_hbm, dst_hbm, zeros_hbm, agg_out,
                idx_s, idx_d, rows, agg_sh, sem):
    cid = lax.axis_index("c")
    sid = lax.axis_index("s")
    wid = sid * NC + cid

    _zero_shared(sid, zeros_hbm, agg_sh)
    plsc.subcore_barrier()

    def chunk(c, carry):
        base = wid * EPW + c * CH
        pltpu.sync_copy(src_hbm.at[pl.ds(base, CH)], idx_s)
        pltpu.sync_copy(dst_hbm.at[pl.ds(base, CH)], idx_d)
        pltpu.async_copy(x_hbm.at[idx_s], rows, sem).wait()
        pltpu.sync_copy(rows, agg_sh.at[idx_d], add=True)
        return carry

    lax.fori_loop(0, NCHUNK, chunk, 0)
    plsc.subcore_barrier()
    _drain_shared(cid, sid, agg_sh, agg_out)


@functools.lru_cache(maxsize=None)
def _make_seg():
    return pl.kernel(
        _seg_kernel,
        out_type=[jax.ShapeDtypeStruct((NC, N, D), jnp.float32)],
        mesh=_get_mesh(),
        scratch_types=[
            pltpu.VMEM((CH,), jnp.int32),
            pltpu.VMEM((CH,), jnp.int32),
            pltpu.VMEM((CH, D), jnp.float32),
            pltpu.VMEM_SHARED((N, D), jnp.float32),
            pltpu.SemaphoreType.DMA,
        ],
    )


def _deg_kernel(dst_hbm, zeros_hbm, ones_hbm, deg_out,
                idx_d, ones_v, deg_sh):
    cid = lax.axis_index("c")
    sid = lax.axis_index("s")
    wid = sid * NC + cid

    _zero_shared(sid, zeros_hbm, deg_sh)
    pltpu.sync_copy(ones_hbm, ones_v)
    plsc.subcore_barrier()

    def chunk(c, carry):
        base = wid * EPW + c * CH
        pltpu.sync_copy(dst_hbm.at[pl.ds(base, CH)], idx_d)
        pltpu.sync_copy(ones_v, deg_sh.at[idx_d], add=True)
        return carry

    lax.fori_loop(0, NCHUNK, chunk, 0)
    plsc.subcore_barrier()
    _drain_shared(cid, sid, deg_sh, deg_out)


@functools.lru_cache(maxsize=None)
def _make_deg():
    return pl.kernel(
        _deg_kernel,
        out_type=[jax.ShapeDtypeStruct((NC, N, D), jnp.float32)],
        mesh=_get_mesh(),
        scratch_types=[
            pltpu.VMEM((CH,), jnp.int32),
            pltpu.VMEM((CH, D), jnp.float32),
            pltpu.VMEM_SHARED((N, D), jnp.float32),
        ],
    )


def _gather_kernel(x_hbm, src_hbm, dst_hbm, xs_out, xd_out,
                   idx_s, idx_d, rows_s, rows_d, sem_s, sem_d):
    cid = lax.axis_index("c")
    sid = lax.axis_index("s")
    wid = sid * NC + cid

    def chunk(c, carry):
        base = wid * EPW + c * CH
        pltpu.sync_copy(src_hbm.at[pl.ds(base, CH)], idx_s)
        pltpu.sync_copy(dst_hbm.at[pl.ds(base, CH)], idx_d)
        a = pltpu.async_copy(x_hbm.at[idx_s], rows_s, sem_s)
        b = pltpu.async_copy(x_hbm.at[idx_d], rows_d, sem_d)
        a.wait()
        b.wait()
        pltpu.sync_copy(rows_s, xs_out.at[pl.ds(base, CH)])
        pltpu.sync_copy(rows_d, xd_out.at[pl.ds(base, CH)])
        return carry

    lax.fori_loop(0, NCHUNK, chunk, 0)


@functools.lru_cache(maxsize=None)
def _make_gather():
    return pl.kernel(
        _gather_kernel,
        out_type=[jax.ShapeDtypeStruct((E, D), jnp.float32),
                  jax.ShapeDtypeStruct((E, D), jnp.float32)],
        mesh=_get_mesh(),
        scratch_types=[
            pltpu.VMEM((CH,), jnp.int32),
            pltpu.VMEM((CH,), jnp.int32),
            pltpu.VMEM((CH, D), jnp.float32),
            pltpu.VMEM((CH, D), jnp.float32),
            pltpu.SemaphoreType.DMA,
            pltpu.SemaphoreType.DMA,
        ],
    )


def _dense0_body(aggp_ref, degp_ref, x_ref, wl_ref, bl_ref, wr_ref, g_ref,
                 b_ref, o_ref, dinv_ref):
    deg = degp_ref[0, :, 0:1] + degp_ref[1, :, 0:1]
    dinv = 1.0 / jnp.maximum(deg, 1.0)
    dinv_ref[...] = dinv
    _dense_common(aggp_ref, dinv, x_ref, wl_ref, bl_ref, wr_ref, g_ref,
                  b_ref, o_ref)


def _dense_body(aggp_ref, dinv_ref, x_ref, wl_ref, bl_ref, wr_ref, g_ref,
                b_ref, o_ref):
    _dense_common(aggp_ref, dinv_ref[...], x_ref, wl_ref, bl_ref, wr_ref,
                  g_ref, b_ref, o_ref)


def _dense_common(aggp_ref, dinv, x_ref, wl_ref, bl_ref, wr_ref, g_ref,
                  b_ref, o_ref):
    agg = (aggp_ref[0] + aggp_ref[1]) * dinv
    out = (jnp.dot(agg, wl_ref[...], preferred_element_type=jnp.float32)
           + bl_ref[...]
           + jnp.dot(x_ref[...], wr_ref[...], preferred_element_type=jnp.float32))
    mean = jnp.mean(out, axis=0, keepdims=True)
    var = jnp.mean((out - mean) * (out - mean), axis=0, keepdims=True)
    out = (out - mean) / jnp.sqrt(var + 1e-5) * g_ref[...] + b_ref[...]
    o_ref[...] = jnp.maximum(out, 0.0) + x_ref[...]


_dense0 = pl.pallas_call(
    _dense0_body,
    out_shape=(jax.ShapeDtypeStruct((N, D), jnp.float32),
               jax.ShapeDtypeStruct((N, 1), jnp.float32)),
)

_dense = pl.pallas_call(
    _dense_body,
    out_shape=jax.ShapeDtypeStruct((N, D), jnp.float32),
)


def _mlp_body(xs_ref, xd_ref, wm1_ref, bm1_ref, wm2_ref, bm2_ref, wm3_ref,
              bm3_ref, o_ref):
    a = (jnp.dot(xs_ref[...], wm1_ref[0:D, :], preferred_element_type=jnp.float32)
         + jnp.dot(xd_ref[...], wm1_ref[D:2 * D, :], preferred_element_type=jnp.float32))
    z = jnp.maximum(a + bm1_ref[...], 0.0)
    h = jnp.maximum(jnp.dot(z, wm2_ref[...], preferred_element_type=jnp.float32)
                    + bm2_ref[...], 0.0)
    o_ref[...] = (jnp.dot(h, wm3_ref[...], preferred_element_type=jnp.float32)
                  + bm3_ref[...])


BLK = 2560
_mlp = pl.pallas_call(
    _mlp_body,
    grid=(E // BLK,),
    in_specs=[
        pl.BlockSpec((BLK, D), lambda i: (i, 0)),
        pl.BlockSpec((BLK, D), lambda i: (i, 0)),
        pl.BlockSpec((2 * D, H), lambda i: (0, 0)),
        pl.BlockSpec((H,), lambda i: (0,)),
        pl.BlockSpec((H, H), lambda i: (0, 0)),
        pl.BlockSpec((H,), lambda i: (0,)),
        pl.BlockSpec((H, 1), lambda i: (0, 0)),
        pl.BlockSpec((1,), lambda i: (0,)),
    ],
    out_specs=pl.BlockSpec((BLK, 1), lambda i: (i, 0)),
    out_shape=jax.ShapeDtypeStruct((E, 1), jnp.float32),
)


def kernel(x, edge_index, Wl0, bl0, Wr0, g0, b0, Wl1, bl1, Wr1, g1, b1,
           Wl2, bl2, Wr2, g2, b2, Wm1, bm1, Wm2, bm2, Wm3, bm3):
    src = edge_index[0]
    dst = edge_index[1]
    zeros = jnp.zeros((RPT, D), jnp.float32)
    ones = jnp.ones((CH, D), jnp.float32)

    (degp,) = _make_deg()(dst, zeros, ones)
    (aggp,) = _make_seg()(x, src, dst, zeros)
    x, dinv = _dense0(aggp, degp, x, Wl0, bl0, Wr0, g0, b0)
    (aggp,) = _make_seg()(x, src, dst, zeros)
    x = _dense(aggp, dinv, x, Wl1, bl1, Wr1, g1, b1)
    (aggp,) = _make_seg()(x, src, dst, zeros)
    x = _dense(aggp, dinv, x, Wl2, bl2, Wr2, g2, b2)

    xs, xd = _make_gather()(x, src, dst)
    return _mlp(xs, xd, Wm1, bm1, Wm2, bm2, Wm3, bm3)


# speedup vs baseline: 4.5914x; 1.4167x over previous
import functools

import jax
import jax.numpy as jnp
from jax import lax
from jax.experimental import pallas as pl
from jax.experimental.pallas import tpu as pltpu
from jax.experimental.pallas import tpu_sc as plsc

N = 10000
E = 320000
D = 128
H = 256

NC = 2
NS = 16
NW = NC * NS
EPW = E // NW
CH = 80
NCHUNK = EPW // CH
RPT = 624
REM = N - NS * RPT


@functools.lru_cache(maxsize=None)
def _get_mesh():
    return plsc.VectorSubcoreMesh(core_axis_name="c", subcore_axis_name="s",
                                  num_cores=NC, num_subcores=NS)


def _zero_shared(sid, zeros_hbm, sh):
    pltpu.sync_copy(zeros_hbm, sh.at[pl.ds(sid * RPT, RPT)])

    @pl.when(sid == NS - 1)
    def _():
        pltpu.sync_copy(zeros_hbm.at[pl.ds(0, REM)], sh.at[pl.ds(NS * RPT, REM)])


def _drain_shared(cid, sid, sh, out):
    pltpu.sync_copy(sh.at[pl.ds(sid * RPT, RPT)],
                    out.at[cid, pl.ds(sid * RPT, RPT)])

    @pl.when(sid == NS - 1)
    def _():
        pltpu.sync_copy(sh.at[pl.ds(NS * RPT, REM)],
                        out.at[cid, pl.ds(NS * RPT, REM)])


def _seg_kernel(x_hbm, src_hbm, dst_hbm, zeros_hbm, agg_out,
                idx_s0, idx_s1, idx_d0, idx_d1, rows0, rows1, agg_sh,
                gsem0, gsem1, ssem0, ssem1):
    cid = lax.axis_index("c")
    sid = lax.axis_index("s")
    wid = sid * NC + cid
    idx_s = (idx_s0, idx_s1)
    idx_d = (idx_d0, idx_d1)
    rows = (rows0, rows1)
    gsem = (gsem0, gsem1)
    ssem = (ssem0, ssem1)

    _zero_shared(sid, zeros_hbm, agg_sh)
    plsc.subcore_barrier()

    def fetch(c, p):
        base = wid * EPW + c * CH
        pltpu.sync_copy(src_hbm.at[pl.ds(base, CH)], idx_s[p])
        pltpu.sync_copy(dst_hbm.at[pl.ds(base, CH)], idx_d[p])
        pltpu.async_copy(x_hbm.at[idx_s[p]], rows[p], gsem[p])

    def flush(p):
        pltpu.make_async_copy(x_hbm.at[idx_s[p]], rows[p], gsem[p]).wait()
        pltpu.async_copy(rows[p], agg_sh.at[idx_d[p]], ssem[p], add=True)

    def swait(p):
        pltpu.make_async_copy(rows[p], agg_sh.at[idx_d[p]], ssem[p]).wait()

    fetch(0, 0)

    def step(k, carry):
        fetch(2 * k + 1, 1)
        flush(0)
        swait(0)
        fetch(2 * k + 2, 0)
        flush(1)
        swait(1)
        return carry

    lax.fori_loop(0, (NCHUNK - 1) // 2, step, 0)
    flush(0)
    swait(0)
    plsc.subcore_barrier()
    _drain_shared(cid, sid, agg_sh, agg_out)


@functools.lru_cache(maxsize=None)
def _make_seg():
    return pl.kernel(
        _seg_kernel,
        out_type=[jax.ShapeDtypeStruct((NC, N, D), jnp.float32)],
        mesh=_get_mesh(),
        scratch_types=[
            pltpu.VMEM((CH,), jnp.int32),
            pltpu.VMEM((CH,), jnp.int32),
            pltpu.VMEM((CH,), jnp.int32),
            pltpu.VMEM((CH,), jnp.int32),
            pltpu.VMEM((CH, D), jnp.float32),
            pltpu.VMEM((CH, D), jnp.float32),
            pltpu.VMEM_SHARED((N, D), jnp.float32),
            pltpu.SemaphoreType.DMA,
            pltpu.SemaphoreType.DMA,
            pltpu.SemaphoreType.DMA,
            pltpu.SemaphoreType.DMA,
        ],
    )


def _deg_kernel(dst_hbm, zeros_hbm, ones_hbm, deg_out,
                idx_d0, idx_d1, ones_v, deg_sh, ssem0, ssem1):
    cid = lax.axis_index("c")
    sid = lax.axis_index("s")
    wid = sid * NC + cid
    idx_d = (idx_d0, idx_d1)
    ssem = (ssem0, ssem1)

    _zero_shared(sid, zeros_hbm, deg_sh)
    pltpu.sync_copy(ones_hbm, ones_v)
    plsc.subcore_barrier()

    def fire(c, p):
        base = wid * EPW + c * CH
        pltpu.sync_copy(dst_hbm.at[pl.ds(base, CH)], idx_d[p])
        pltpu.async_copy(ones_v, deg_sh.at[idx_d[p]], ssem[p], add=True)

    def swait(p):
        pltpu.make_async_copy(ones_v, deg_sh.at[idx_d[p]], ssem[p]).wait()

    fire(0, 0)

    def step(k, carry):
        fire(2 * k + 1, 1)
        swait(0)
        fire(2 * k + 2, 0)
        swait(1)
        return carry

    lax.fori_loop(0, (NCHUNK - 1) // 2, step, 0)
    swait(0)
    plsc.subcore_barrier()
    _drain_shared(cid, sid, deg_sh, deg_out)


@functools.lru_cache(maxsize=None)
def _make_deg():
    return pl.kernel(
        _deg_kernel,
        out_type=[jax.ShapeDtypeStruct((NC, N, D), jnp.float32)],
        mesh=_get_mesh(),
        scratch_types=[
            pltpu.VMEM((CH,), jnp.int32),
            pltpu.VMEM((CH,), jnp.int32),
            pltpu.VMEM((CH, D), jnp.float32),
            pltpu.VMEM_SHARED((N, D), jnp.float32),
            pltpu.SemaphoreType.DMA,
            pltpu.SemaphoreType.DMA,
        ],
    )


def _gather_kernel(x_hbm, src_hbm, dst_hbm, xs_out, xd_out,
                   idx_s0, idx_s1, idx_d0, idx_d1,
                   rows_s0, rows_s1, rows_d0, rows_d1,
                   gs0, gs1, gd0, gd1, ws0, ws1, wd0, wd1):
    cid = lax.axis_index("c")
    sid = lax.axis_index("s")
    wid = sid * NC + cid
    idx_s = (idx_s0, idx_s1)
    idx_d = (idx_d0, idx_d1)
    rows_s = (rows_s0, rows_s1)
    rows_d = (rows_d0, rows_d1)
    gs = (gs0, gs1)
    gd = (gd0, gd1)
    ws = (ws0, ws1)
    wd = (wd0, wd1)

    def fetch(c, p):
        base = wid * EPW + c * CH
        pltpu.sync_copy(src_hbm.at[pl.ds(base, CH)], idx_s[p])
        pltpu.sync_copy(dst_hbm.at[pl.ds(base, CH)], idx_d[p])
        pltpu.async_copy(x_hbm.at[idx_s[p]], rows_s[p], gs[p])
        pltpu.async_copy(x_hbm.at[idx_d[p]], rows_d[p], gd[p])

    def flush(c, p):
        base = wid * EPW + c * CH
        pltpu.make_async_copy(x_hbm.at[idx_s[p]], rows_s[p], gs[p]).wait()
        pltpu.make_async_copy(x_hbm.at[idx_d[p]], rows_d[p], gd[p]).wait()
        pltpu.async_copy(rows_s[p], xs_out.at[pl.ds(base, CH)], ws[p])
        pltpu.async_copy(rows_d[p], xd_out.at[pl.ds(base, CH)], wd[p])

    def wwait(c, p):
        base = wid * EPW + c * CH
        pltpu.make_async_copy(rows_s[p], xs_out.at[pl.ds(base, CH)], ws[p]).wait()
        pltpu.make_async_copy(rows_d[p], xd_out.at[pl.ds(base, CH)], wd[p]).wait()

    fetch(0, 0)

    def step(k, carry):
        fetch(2 * k + 1, 1)
        flush(2 * k, 0)
        wwait(2 * k, 0)
        fetch(2 * k + 2, 0)
        flush(2 * k + 1, 1)
        wwait(2 * k + 1, 1)
        return carry

    lax.fori_loop(0, (NCHUNK - 1) // 2, step, 0)
    flush(NCHUNK - 1, 0)
    wwait(NCHUNK - 1, 0)


@functools.lru_cache(maxsize=None)
def _make_gather():
    return pl.kernel(
        _gather_kernel,
        out_type=[jax.ShapeDtypeStruct((E, D), jnp.float32),
                  jax.ShapeDtypeStruct((E, D), jnp.float32)],
        mesh=_get_mesh(),
        scratch_types=[
            pltpu.VMEM((CH,), jnp.int32),
            pltpu.VMEM((CH,), jnp.int32),
            pltpu.VMEM((CH,), jnp.int32),
            pltpu.VMEM((CH,), jnp.int32),
            pltpu.VMEM((CH, D), jnp.float32),
            pltpu.VMEM((CH, D), jnp.float32),
            pltpu.VMEM((CH, D), jnp.float32),
            pltpu.VMEM((CH, D), jnp.float32),
            pltpu.SemaphoreType.DMA,
            pltpu.SemaphoreType.DMA,
            pltpu.SemaphoreType.DMA,
            pltpu.SemaphoreType.DMA,
            pltpu.SemaphoreType.DMA,
            pltpu.SemaphoreType.DMA,
            pltpu.SemaphoreType.DMA,
            pltpu.SemaphoreType.DMA,
        ],
    )


def _dense0_body(aggp_ref, degp_ref, x_ref, wl_ref, bl_ref, wr_ref, g_ref,
                 b_ref, o_ref, dinv_ref):
    deg = degp_ref[0, :, 0:1] + degp_ref[1, :, 0:1]
    dinv = 1.0 / jnp.maximum(deg, 1.0)
    dinv_ref[...] = dinv
    _dense_common(aggp_ref, dinv, x_ref, wl_ref, bl_ref, wr_ref, g_ref,
                  b_ref, o_ref)


def _dense_body(aggp_ref, dinv_ref, x_ref, wl_ref, bl_ref, wr_ref, g_ref,
                b_ref, o_ref):
    _dense_common(aggp_ref, dinv_ref[...], x_ref, wl_ref, bl_ref, wr_ref,
                  g_ref, b_ref, o_ref)


def _dense_common(aggp_ref, dinv, x_ref, wl_ref, bl_ref, wr_ref, g_ref,
                  b_ref, o_ref):
    agg = (aggp_ref[0] + aggp_ref[1]) * dinv
    out = (jnp.dot(agg, wl_ref[...], preferred_element_type=jnp.float32)
           + bl_ref[...]
           + jnp.dot(x_ref[...], wr_ref[...], preferred_element_type=jnp.float32))
    mean = jnp.mean(out, axis=0, keepdims=True)
    var = jnp.mean((out - mean) * (out - mean), axis=0, keepdims=True)
    out = (out - mean) / jnp.sqrt(var + 1e-5) * g_ref[...] + b_ref[...]
    o_ref[...] = jnp.maximum(out, 0.0) + x_ref[...]


_dense0 = pl.pallas_call(
    _dense0_body,
    out_shape=(jax.ShapeDtypeStruct((N, D), jnp.float32),
               jax.ShapeDtypeStruct((N, 1), jnp.float32)),
)

_dense = pl.pallas_call(
    _dense_body,
    out_shape=jax.ShapeDtypeStruct((N, D), jnp.float32),
)


def _mlp_body(xs_ref, xd_ref, wm1_ref, bm1_ref, wm2_ref, bm2_ref, wm3_ref,
              bm3_ref, o_ref):
    a = (jnp.dot(xs_ref[...], wm1_ref[0:D, :], preferred_element_type=jnp.float32)
         + jnp.dot(xd_ref[...], wm1_ref[D:2 * D, :], preferred_element_type=jnp.float32))
    z = jnp.maximum(a + bm1_ref[...], 0.0)
    h = jnp.maximum(jnp.dot(z, wm2_ref[...], preferred_element_type=jnp.float32)
                    + bm2_ref[...], 0.0)
    o_ref[...] = (jnp.dot(h, wm3_ref[...], preferred_element_type=jnp.float32)
                  + bm3_ref[...])


BLK = 2560
_mlp = pl.pallas_call(
    _mlp_body,
    grid=(E // BLK,),
    in_specs=[
        pl.BlockSpec((BLK, D), lambda i: (i, 0)),
        pl.BlockSpec((BLK, D), lambda i: (i, 0)),
        pl.BlockSpec((2 * D, H), lambda i: (0, 0)),
        pl.BlockSpec((H,), lambda i: (0,)),
        pl.BlockSpec((H, H), lambda i: (0, 0)),
        pl.BlockSpec((H,), lambda i: (0,)),
        pl.BlockSpec((H, 1), lambda i: (0, 0)),
        pl.BlockSpec((1,), lambda i: (0,)),
    ],
    out_specs=pl.BlockSpec((BLK, 1), lambda i: (i, 0)),
    out_shape=jax.ShapeDtypeStruct((E, 1), jnp.float32),
)


def kernel(x, edge_index, Wl0, bl0, Wr0, g0, b0, Wl1, bl1, Wr1, g1, b1,
           Wl2, bl2, Wr2, g2, b2, Wm1, bm1, Wm2, bm2, Wm3, bm3):
    src = edge_index[0]
    dst = edge_index[1]
    zeros = jnp.zeros((RPT, D), jnp.float32)
    ones = jnp.ones((CH, D), jnp.float32)

    (degp,) = _make_deg()(dst, zeros, ones)
    (aggp,) = _make_seg()(x, src, dst, zeros)
    x, dinv = _dense0(aggp, degp, x, Wl0, bl0, Wr0, g0, b0)
    (aggp,) = _make_seg()(x, src, dst, zeros)
    x = _dense(aggp, dinv, x, Wl1, bl1, Wr1, g1, b1)
    (aggp,) = _make_seg()(x, src, dst, zeros)
    x = _dense(aggp, dinv, x, Wl2, bl2, Wr2, g2, b2)

    xs, xd = _make_gather()(x, src, dst)
    return _mlp(xs, xd, Wm1, bm1, Wm2, bm2, Wm3, bm3)
